# baseline (speedup 1.0000x reference)
import functools

import jax
import jax.numpy as jnp
from jax import lax
from jax.experimental import pallas as pl
from jax.experimental.pallas import tpu as pltpu
from jax.experimental.pallas import tpu_sc as plsc

N_NODES = 10000
N_EDGES = 320000
D = 128

NUM_SC = 2
NUM_SUB = 16
NW = NUM_SC * NUM_SUB
CHUNK = 80
NB_G = 5
NB_S = 4

NC_PHASES = (63, 62)
E_PHASES = tuple(NW * CHUNK * nc for nc in NC_PHASES)
N_PHASES = len(NC_PHASES)
assert sum(E_PHASES) == N_EDGES

EDGE_BLOCK = 2560
NODE_BLOCK = 2000

N_PER_SUB = 624
N_LAST_SUB = N_NODES - (NUM_SUB - 1) * N_PER_SUB

_sc_mesh = plsc.VectorSubcoreMesh(core_axis_name="c", subcore_axis_name="s")


def _make_gather_body(nc, eoff):
    def body(node_hbm, src_hbm, dst_hbm, gs_hbm, gd_hbm,
             idxs, idxd, rows_s, rows_d,
             isem_s, isem_d, gsem_s, gsem_d, ssem_s, ssem_d):
        wid = lax.axis_index("c") * NUM_SUB + lax.axis_index("s")
        base0 = wid * nc * CHUNK

        def fire_idx(t, b):
            base = eoff + base0 + t * CHUNK
            pltpu.async_copy(src_hbm.at[pl.ds(base, CHUNK)], idxs.at[b],
                             isem_s.at[b])
            pltpu.async_copy(dst_hbm.at[pl.ds(base, CHUNK)], idxd.at[b],
                             isem_d.at[b])

        def wait_idx(b):
            pltpu.make_async_copy(src_hbm.at[pl.ds(base0, CHUNK)], idxs.at[b],
                                  isem_s.at[b]).wait()
            pltpu.make_async_copy(dst_hbm.at[pl.ds(base0, CHUNK)], idxd.at[b],
                                  isem_d.at[b]).wait()

        def fire_gather(b):
            pltpu.async_copy(node_hbm.at[idxs.at[b]], rows_s.at[b],
                             gsem_s.at[b])
            pltpu.async_copy(node_hbm.at[idxd.at[b]], rows_d.at[b],
                             gsem_d.at[b])

        def wait_gather(b):
            pltpu.make_async_copy(node_hbm.at[idxs.at[b]], rows_s.at[b],
                                  gsem_s.at[b]).wait()
            pltpu.make_async_copy(node_hbm.at[idxd.at[b]], rows_d.at[b],
                                  gsem_d.at[b]).wait()

        def store(t, b):
            base = base0 + t * CHUNK
            pltpu.async_copy(rows_s.at[b], gs_hbm.at[pl.ds(base, CHUNK)],
                             ssem_s.at[b])
            pltpu.async_copy(rows_d.at[b], gd_hbm.at[pl.ds(base, CHUNK)],
                             ssem_d.at[b])

        def wait_store(b):
            pltpu.make_async_copy(rows_s.at[b], gs_hbm.at[pl.ds(base0, CHUNK)],
                                  ssem_s.at[b]).wait()
            pltpu.make_async_copy(rows_d.at[b], gd_hbm.at[pl.ds(base0, CHUNK)],
                                  ssem_d.at[b]).wait()

        @pl.loop(0, nc + 2)
        def _(t):
            @pl.when(t < nc)
            def _():
                fire_idx(t, lax.rem(t, NB_G))

            g = t - 1

            @pl.when(jnp.logical_and(g >= 0, g < nc))
            def _():
                bg = lax.rem(g, NB_G)
                wait_idx(bg)

                @pl.when(g >= NB_G)
                def _():
                    wait_store(bg)

                fire_gather(bg)

            c = t - 2

            @pl.when(c >= 0)
            def _():
                bc = lax.rem(c, NB_G)
                wait_gather(bc)
                store(c, bc)

        for b in range(NB_G):
            wait_store(b)

    return body


def _sc_gather(node_rep, src, dst, nc, eoff):
    e_phase = NW * CHUNK * nc
    fn = pl.kernel(
        _make_gather_body(nc, eoff),
        out_type=(jax.ShapeDtypeStruct((e_phase, D), jnp.float32),
                  jax.ShapeDtypeStruct((e_phase, D), jnp.float32)),
        mesh=_sc_mesh,
        scratch_types=[
            pltpu.VMEM((NB_G, CHUNK), jnp.int32),
            pltpu.VMEM((NB_G, CHUNK), jnp.int32),
            pltpu.VMEM((NB_G, CHUNK, D), jnp.float32),
            pltpu.VMEM((NB_G, CHUNK, D), jnp.float32),
            pltpu.SemaphoreType.DMA((NB_G,)),
            pltpu.SemaphoreType.DMA((NB_G,)),
            pltpu.SemaphoreType.DMA((NB_G,)),
            pltpu.SemaphoreType.DMA((NB_G,)),
            pltpu.SemaphoreType.DMA((NB_G,)),
            pltpu.SemaphoreType.DMA((NB_G,)),
        ],
    )
    return fn(node_rep, src, dst)


def _make_scatter_body(nc, eoff):
    def body(eo_hbm, src_hbm, dst_hbm, zeros_hbm, part_hbm,
             idxs, idxd, rows, acc, isem_s, isem_d, gsem, asem_s, asem_d):
        c = lax.axis_index("c")
        s = lax.axis_index("s")
        wid = c * NUM_SUB + s
        base0 = wid * nc * CHUNK

        @pl.when(s < NUM_SUB - 1)
        def _():
            pltpu.sync_copy(zeros_hbm.at[pl.ds(s * N_PER_SUB, N_PER_SUB)],
                            acc.at[pl.ds(s * N_PER_SUB, N_PER_SUB)])

        @pl.when(s == NUM_SUB - 1)
        def _():
            pltpu.sync_copy(zeros_hbm.at[pl.ds(s * N_PER_SUB, N_LAST_SUB)],
                            acc.at[pl.ds(s * N_PER_SUB, N_LAST_SUB)])

        plsc.subcore_barrier()

        def fire_loads(t, b):
            base = base0 + t * CHUNK
            pltpu.async_copy(src_hbm.at[pl.ds(eoff + base, CHUNK)], idxs.at[b],
                             isem_s.at[b])
            pltpu.async_copy(dst_hbm.at[pl.ds(eoff + base, CHUNK)], idxd.at[b],
                             isem_d.at[b])
            pltpu.async_copy(eo_hbm.at[pl.ds(base, CHUNK)], rows.at[b],
                             gsem.at[b])

        def wait_loads(b):
            pltpu.make_async_copy(src_hbm.at[pl.ds(base0, CHUNK)], idxs.at[b],
                                  isem_s.at[b]).wait()
            pltpu.make_async_copy(dst_hbm.at[pl.ds(base0, CHUNK)], idxd.at[b],
                                  isem_d.at[b]).wait()
            pltpu.make_async_copy(eo_hbm.at[pl.ds(base0, CHUNK)], rows.at[b],
                                  gsem.at[b]).wait()

        def fire_adds(b):
            pltpu.async_copy(rows.at[b], acc.at[idxs.at[b]], asem_s.at[b],
                             add=True)
            pltpu.async_copy(rows.at[b], acc.at[idxd.at[b]], asem_d.at[b],
                             add=True)

        def wait_adds(b):
            pltpu.make_async_copy(rows.at[b], acc.at[idxs.at[b]],
                                  asem_s.at[b]).wait()
            pltpu.make_async_copy(rows.at[b], acc.at[idxd.at[b]],
                                  asem_d.at[b]).wait()

        @pl.loop(0, nc + NB_S - 1)
        def _(t):
            @pl.when(t < nc)
            def _():
                b = lax.rem(t, NB_S)

                @pl.when(t >= NB_S)
                def _():
                    wait_adds(b)

                fire_loads(t, b)

            comp = t - (NB_S - 1)

            @pl.when(comp >= 0)
            def _():
                bc = lax.rem(comp, NB_S)
                wait_loads(bc)
                fire_adds(bc)

        for b in range(NB_S):
            wait_adds(b)

        plsc.subcore_barrier()

        @pl.when(s < NUM_SUB - 1)
        def _():
            pltpu.sync_copy(acc.at[pl.ds(s * N_PER_SUB, N_PER_SUB)],
                            part_hbm.at[c].at[pl.ds(s * N_PER_SUB, N_PER_SUB)])

        @pl.when(s == NUM_SUB - 1)
        def _():
            pltpu.sync_copy(acc.at[pl.ds(s * N_PER_SUB, N_LAST_SUB)],
                            part_hbm.at[c].at[pl.ds(s * N_PER_SUB, N_LAST_SUB)])

    return body


def _sc_scatter(edge_out_slice, src, dst, zeros, nc, eoff):
    fn = pl.kernel(
        _make_scatter_body(nc, eoff),
        out_type=jax.ShapeDtypeStruct((NUM_SC, N_NODES, D), jnp.float32),
        mesh=_sc_mesh,
        scratch_types=[
            pltpu.VMEM((NB_S, CHUNK), jnp.int32),
            pltpu.VMEM((NB_S, CHUNK), jnp.int32),
            pltpu.VMEM((NB_S, CHUNK, D), jnp.float32),
            pltpu.VMEM_SHARED((N_NODES, D), jnp.float32),
            pltpu.SemaphoreType.DMA((NB_S,)),
            pltpu.SemaphoreType.DMA((NB_S,)),
            pltpu.SemaphoreType.DMA((NB_S,)),
            pltpu.SemaphoreType.DMA((NB_S,)),
            pltpu.SemaphoreType.DMA((NB_S,)),
        ],
    )
    return fn(edge_out_slice, src, dst, zeros)


def _edge_mlp_body(er_ref, gs_ref, gd_ref, w1_ref, b1_ref, w2_ref, b2_ref,
                   out_ref):
    n2e = (gs_ref[...] + gd_ref[...]).astype(jnp.bfloat16)
    x = jnp.concatenate([er_ref[...].astype(jnp.bfloat16), n2e], axis=-1)
    h = jnp.dot(x, w1_ref[...].astype(jnp.bfloat16),
                preferred_element_type=jnp.float32)
    h = jnp.maximum(h + b1_ref[...], 0.0).astype(jnp.bfloat16)
    o = jnp.dot(h, w2_ref[...].astype(jnp.bfloat16),
                preferred_element_type=jnp.float32)
    out_ref[...] = jnp.maximum(o + b2_ref[...], 0.0)


def _edge_mlp_phase(edge_rep, gs_p, gd_p, W1, bias1, W2, bias2, blk_off):
    n = gs_p.shape[0]
    row = lambda i: (i, 0)
    er_map = lambda i: (i + blk_off, 0)
    full = lambda i: (0, 0)
    return pl.pallas_call(
        _edge_mlp_body,
        grid=(n // EDGE_BLOCK,),
        in_specs=[
            pl.BlockSpec((EDGE_BLOCK, D), er_map),
            pl.BlockSpec((EDGE_BLOCK, D), row),
            pl.BlockSpec((EDGE_BLOCK, D), row),
            pl.BlockSpec((2 * D, 2 * D), full),
            pl.BlockSpec((1, 2 * D), full),
            pl.BlockSpec((2 * D, D), full),
            pl.BlockSpec((1, D), full),
        ],
        out_specs=pl.BlockSpec((EDGE_BLOCK, D), row),
        out_shape=jax.ShapeDtypeStruct((n, D), jnp.float32),
    )(edge_rep, gs_p, gd_p, W1, bias1.reshape(1, -1), W2, bias2.reshape(1, -1))


N_PART = 2 * N_PHASES


def _node_mlp_body(*refs):
    nr_ref = refs[0]
    extras = refs[1:1 + N_PART]
    w1_ref, b1_ref, w2_ref, b2_ref, out_ref = refs[1 + N_PART:]
    e2n = extras[0][...]
    for e in extras[1:]:
        e2n = e2n + e[...]
    x = jnp.concatenate([nr_ref[...].astype(jnp.bfloat16),
                         e2n.astype(jnp.bfloat16)], axis=-1)
    h = jnp.dot(x, w1_ref[...].astype(jnp.bfloat16),
                preferred_element_type=jnp.float32)
    h = jnp.maximum(h + b1_ref[...], 0.0).astype(jnp.bfloat16)
    o = jnp.dot(h, w2_ref[...].astype(jnp.bfloat16),
                preferred_element_type=jnp.float32)
    out_ref[...] = jnp.maximum(o + b2_ref[...], 0.0)


def _node_mlp(node_rep, partials, W1, bias1, W2, bias2):
    row = lambda i: (i, 0)
    full = lambda i: (0, 0)
    return pl.pallas_call(
        _node_mlp_body,
        grid=(N_NODES // NODE_BLOCK,),
        in_specs=[pl.BlockSpec((NODE_BLOCK, D), row)] * (1 + N_PART) + [
            pl.BlockSpec((2 * D, 2 * D), full),
            pl.BlockSpec((1, 2 * D), full),
            pl.BlockSpec((2 * D, D), full),
            pl.BlockSpec((1, D), full),
        ],
        out_specs=pl.BlockSpec((NODE_BLOCK, D), row),
        out_shape=jax.ShapeDtypeStruct((N_NODES, D), jnp.float32),
    )(node_rep, *partials, W1, bias1.reshape(1, -1), W2, bias2.reshape(1, -1))


def _copy_first_body(src_ref, out_ref):
    out_ref[...] = src_ref[...]


def _copy_next_body(prev_ref, src_ref, out_ref):
    out_ref[...] = src_ref[...]


def _assemble_edge_out(prev, eo_p, blk_off):
    nblk = eo_p.shape[0] // EDGE_BLOCK
    row = lambda i: (i, 0)
    out_map = lambda i: (i + blk_off, 0)
    out_shape = jax.ShapeDtypeStruct((N_EDGES, D), jnp.float32)
    if prev is None:
        return pl.pallas_call(
            _copy_first_body,
            grid=(nblk,),
            in_specs=[pl.BlockSpec((EDGE_BLOCK, D), row)],
            out_specs=pl.BlockSpec((EDGE_BLOCK, D), out_map),
            out_shape=out_shape,
        )(eo_p)
    return pl.pallas_call(
        _copy_next_body,
        grid=(nblk,),
        in_specs=[
            pl.BlockSpec((8, D), lambda i: (0, 0)),
            pl.BlockSpec((EDGE_BLOCK, D), row),
        ],
        out_specs=pl.BlockSpec((EDGE_BLOCK, D), out_map),
        out_shape=out_shape,
        input_output_aliases={0: 0},
    )(prev, eo_p)


def kernel(node_rep, edge_rep, edge_index, We1, be1, We2, be2, Wn1, bn1, Wn2, bn2):
    src = edge_index[0]
    dst = edge_index[1]
    zeros = jnp.zeros((N_NODES, D), jnp.float32)

    bounds = [0]
    for e in E_PHASES:
        bounds.append(bounds[-1] + e)

    gathered = [_sc_gather(node_rep, src, dst, NC_PHASES[p], bounds[p])
                for p in range(N_PHASES)]

    eo_slices = []
    for p in range(N_PHASES):
        gs_p, gd_p = gathered[p]
        eo_slices.append(_edge_mlp_phase(edge_rep, gs_p, gd_p, We1, be1,
                                         We2, be2, bounds[p] // EDGE_BLOCK))

    partials = []
    edge_out = None
    for p in range(N_PHASES):
        part = _sc_scatter(eo_slices[p], src, dst, zeros, NC_PHASES[p],
                           bounds[p])
        partials.extend([part[0], part[1]])
        edge_out = _assemble_edge_out(edge_out, eo_slices[p],
                                      bounds[p] // EDGE_BLOCK)

    node_out = _node_mlp(node_rep, partials, Wn1, bn1, Wn2, bn2)
    return (node_out, edge_out)


# speedup vs baseline: 1.2205x; 1.2205x over previous
import functools

import jax
import jax.numpy as jnp
from jax import lax
from jax.experimental import pallas as pl
from jax.experimental.pallas import tpu as pltpu
from jax.experimental.pallas import tpu_sc as plsc

N_NODES = 10000
N_EDGES = 320000
D = 128

NUM_SC = 2
NUM_SUB = 16
NW = NUM_SC * NUM_SUB
CHUNK = 80
NB_G = 4
NB_S = 4

NC_PHASES = (63, 62)
E_PHASES = tuple(NW * CHUNK * nc for nc in NC_PHASES)
N_PHASES = len(NC_PHASES)
assert sum(E_PHASES) == N_EDGES

EDGE_BLOCK = 2560
NODE_BLOCK = 2000

N_PER_SUB = 624
N_LAST_SUB = N_NODES - (NUM_SUB - 1) * N_PER_SUB

_sc_mesh = plsc.VectorSubcoreMesh(core_axis_name="c", subcore_axis_name="s")


S_SLOTS = NUM_SUB * NB_G * CHUNK


def _make_gather_body(nc, eoff):
    def body(node_hbm, src_hbm, dst_hbm, ident_hbm, n2e_hbm,
             idxs, idxd, rows_a, rows_b, ival, stage,
             isem_s, isem_d, gsem_a, gsem_b, csem, asem, ssem):
        s = lax.axis_index("s")
        wid = lax.axis_index("c") * NUM_SUB + s
        base0 = wid * nc * CHUNK

        for b in range(NB_G):
            pltpu.sync_copy(
                ident_hbm.at[pl.ds(s * NB_G * CHUNK + b * CHUNK, CHUNK)],
                ival.at[b])

        def slot(b):
            return pl.multiple_of((s * NB_G + b) * CHUNK, 8)

        def fire_idx(t, b):
            base = eoff + base0 + t * CHUNK
            pltpu.async_copy(src_hbm.at[pl.ds(base, CHUNK)], idxs.at[b],
                             isem_s.at[b])
            pltpu.async_copy(dst_hbm.at[pl.ds(base, CHUNK)], idxd.at[b],
                             isem_d.at[b])

        def wait_idx(b):
            pltpu.make_async_copy(src_hbm.at[pl.ds(base0, CHUNK)], idxs.at[b],
                                  isem_s.at[b]).wait()
            pltpu.make_async_copy(dst_hbm.at[pl.ds(base0, CHUNK)], idxd.at[b],
                                  isem_d.at[b]).wait()

        def fire_gather(b):
            pltpu.async_copy(node_hbm.at[idxs.at[b]], rows_a.at[b],
                             gsem_a.at[b])
            pltpu.async_copy(node_hbm.at[idxd.at[b]], rows_b.at[b],
                             gsem_b.at[b])

        def wait_gather(b):
            pltpu.make_async_copy(node_hbm.at[idxs.at[b]], rows_a.at[b],
                                  gsem_a.at[b]).wait()
            pltpu.make_async_copy(node_hbm.at[idxd.at[b]], rows_b.at[b],
                                  gsem_b.at[b]).wait()

        def fire_copy(b):
            pltpu.async_copy(rows_a.at[b], stage.at[pl.ds(slot(b), CHUNK)],
                             csem.at[b])

        def wait_copy(b):
            pltpu.make_async_copy(rows_a.at[b],
                                  stage.at[pl.ds(slot(b), CHUNK)],
                                  csem.at[b]).wait()

        def fire_add(b):
            pltpu.async_copy(rows_b.at[b], stage.at[ival.at[b]], asem.at[b],
                             add=True)

        def wait_add(b):
            pltpu.make_async_copy(rows_b.at[b], stage.at[ival.at[b]],
                                  asem.at[b]).wait()

        def fire_store(t, b):
            base = base0 + t * CHUNK
            pltpu.async_copy(stage.at[pl.ds(slot(b), CHUNK)],
                             n2e_hbm.at[pl.ds(base, CHUNK)], ssem.at[b])

        def wait_store(b):
            pltpu.make_async_copy(stage.at[pl.ds(slot(b), CHUNK)],
                                  n2e_hbm.at[pl.ds(base0, CHUNK)],
                                  ssem.at[b]).wait()

        @pl.loop(0, nc + 4)
        def _(t):
            @pl.when(t < nc)
            def _():
                fire_idx(t, lax.rem(t, NB_G))

            u = t - 1

            @pl.when(jnp.logical_and(u >= 0, u < nc))
            def _():
                bu = lax.rem(u, NB_G)
                wait_idx(bu)
                fire_gather(bu)

            v = t - 2

            @pl.when(jnp.logical_and(v >= 0, v < nc))
            def _():
                bv = lax.rem(v, NB_G)
                wait_gather(bv)

                @pl.when(v >= NB_G)
                def _():
                    wait_store(bv)

                fire_copy(bv)

            w = t - 3

            @pl.when(jnp.logical_and(w >= 0, w < nc))
            def _():
                bw = lax.rem(w, NB_G)
                wait_copy(bw)
                fire_add(bw)

            x = t - 4

            @pl.when(jnp.logical_and(x >= 0, x < nc))
            def _():
                bx = lax.rem(x, NB_G)
                wait_add(bx)
                fire_store(x, bx)

        for b in range(NB_G):
            wait_store(b)

    return body


def _sc_gather(node_rep, src, dst, ident, nc, eoff):
    e_phase = NW * CHUNK * nc
    fn = pl.kernel(
        _make_gather_body(nc, eoff),
        out_type=jax.ShapeDtypeStruct((e_phase, D), jnp.float32),
        mesh=_sc_mesh,
        scratch_types=[
            pltpu.VMEM((NB_G, CHUNK), jnp.int32),
            pltpu.VMEM((NB_G, CHUNK), jnp.int32),
            pltpu.VMEM((NB_G, CHUNK, D), jnp.float32),
            pltpu.VMEM((NB_G, CHUNK, D), jnp.float32),
            pltpu.VMEM((NB_G, CHUNK), jnp.int32),
            pltpu.VMEM_SHARED((S_SLOTS, D), jnp.float32),
            pltpu.SemaphoreType.DMA((NB_G,)),
            pltpu.SemaphoreType.DMA((NB_G,)),
            pltpu.SemaphoreType.DMA((NB_G,)),
            pltpu.SemaphoreType.DMA((NB_G,)),
            pltpu.SemaphoreType.DMA((NB_G,)),
            pltpu.SemaphoreType.DMA((NB_G,)),
            pltpu.SemaphoreType.DMA((NB_G,)),
        ],
    )
    return fn(node_rep, src, dst, ident)


def _make_scatter_body(nc, eoff):
    def body(eo_hbm, src_hbm, dst_hbm, zeros_hbm, part_hbm,
             idxs, idxd, rows, acc, isem_s, isem_d, gsem, asem_s, asem_d):
        c = lax.axis_index("c")
        s = lax.axis_index("s")
        wid = c * NUM_SUB + s
        base0 = wid * nc * CHUNK

        @pl.when(s < NUM_SUB - 1)
        def _():
            pltpu.sync_copy(zeros_hbm.at[pl.ds(s * N_PER_SUB, N_PER_SUB)],
                            acc.at[pl.ds(s * N_PER_SUB, N_PER_SUB)])

        @pl.when(s == NUM_SUB - 1)
        def _():
            pltpu.sync_copy(zeros_hbm.at[pl.ds(s * N_PER_SUB, N_LAST_SUB)],
                            acc.at[pl.ds(s * N_PER_SUB, N_LAST_SUB)])

        plsc.subcore_barrier()

        def fire_loads(t, b):
            base = base0 + t * CHUNK
            pltpu.async_copy(src_hbm.at[pl.ds(eoff + base, CHUNK)], idxs.at[b],
                             isem_s.at[b])
            pltpu.async_copy(dst_hbm.at[pl.ds(eoff + base, CHUNK)], idxd.at[b],
                             isem_d.at[b])
            pltpu.async_copy(eo_hbm.at[pl.ds(base, CHUNK)], rows.at[b],
                             gsem.at[b])

        def wait_loads(b):
            pltpu.make_async_copy(src_hbm.at[pl.ds(base0, CHUNK)], idxs.at[b],
                                  isem_s.at[b]).wait()
            pltpu.make_async_copy(dst_hbm.at[pl.ds(base0, CHUNK)], idxd.at[b],
                                  isem_d.at[b]).wait()
            pltpu.make_async_copy(eo_hbm.at[pl.ds(base0, CHUNK)], rows.at[b],
                                  gsem.at[b]).wait()

        def fire_adds(b):
            pltpu.async_copy(rows.at[b], acc.at[idxs.at[b]], asem_s.at[b],
                             add=True)
            pltpu.async_copy(rows.at[b], acc.at[idxd.at[b]], asem_d.at[b],
                             add=True)

        def wait_adds(b):
            pltpu.make_async_copy(rows.at[b], acc.at[idxs.at[b]],
                                  asem_s.at[b]).wait()
            pltpu.make_async_copy(rows.at[b], acc.at[idxd.at[b]],
                                  asem_d.at[b]).wait()

        @pl.loop(0, nc + NB_S - 1)
        def _(t):
            @pl.when(t < nc)
            def _():
                b = lax.rem(t, NB_S)

                @pl.when(t >= NB_S)
                def _():
                    wait_adds(b)

                fire_loads(t, b)

            comp = t - (NB_S - 1)

            @pl.when(comp >= 0)
            def _():
                bc = lax.rem(comp, NB_S)
                wait_loads(bc)
                fire_adds(bc)

        for b in range(NB_S):
            wait_adds(b)

        plsc.subcore_barrier()

        @pl.when(s < NUM_SUB - 1)
        def _():
            pltpu.sync_copy(acc.at[pl.ds(s * N_PER_SUB, N_PER_SUB)],
                            part_hbm.at[c].at[pl.ds(s * N_PER_SUB, N_PER_SUB)])

        @pl.when(s == NUM_SUB - 1)
        def _():
            pltpu.sync_copy(acc.at[pl.ds(s * N_PER_SUB, N_LAST_SUB)],
                            part_hbm.at[c].at[pl.ds(s * N_PER_SUB, N_LAST_SUB)])

    return body


def _sc_scatter(edge_out_slice, src, dst, zeros, nc, eoff):
    fn = pl.kernel(
        _make_scatter_body(nc, eoff),
        out_type=jax.ShapeDtypeStruct((NUM_SC, N_NODES, D), jnp.float32),
        mesh=_sc_mesh,
        scratch_types=[
            pltpu.VMEM((NB_S, CHUNK), jnp.int32),
            pltpu.VMEM((NB_S, CHUNK), jnp.int32),
            pltpu.VMEM((NB_S, CHUNK, D), jnp.float32),
            pltpu.VMEM_SHARED((N_NODES, D), jnp.float32),
            pltpu.SemaphoreType.DMA((NB_S,)),
            pltpu.SemaphoreType.DMA((NB_S,)),
            pltpu.SemaphoreType.DMA((NB_S,)),
            pltpu.SemaphoreType.DMA((NB_S,)),
            pltpu.SemaphoreType.DMA((NB_S,)),
        ],
    )
    return fn(edge_out_slice, src, dst, zeros)


def _edge_mlp_body(er_ref, n2e_ref, w1_ref, b1_ref, w2_ref, b2_ref,
                   out_ref):
    x = jnp.concatenate([er_ref[...].astype(jnp.bfloat16),
                         n2e_ref[...].astype(jnp.bfloat16)], axis=-1)
    h = jnp.dot(x, w1_ref[...].astype(jnp.bfloat16),
                preferred_element_type=jnp.float32)
    h = jnp.maximum(h + b1_ref[...], 0.0).astype(jnp.bfloat16)
    o = jnp.dot(h, w2_ref[...].astype(jnp.bfloat16),
                preferred_element_type=jnp.float32)
    out_ref[...] = jnp.maximum(o + b2_ref[...], 0.0)


def _edge_mlp_phase(edge_rep, n2e_p, W1, bias1, W2, bias2, blk_off):
    n = n2e_p.shape[0]
    row = lambda i: (i, 0)
    er_map = lambda i: (i + blk_off, 0)
    full = lambda i: (0, 0)
    return pl.pallas_call(
        _edge_mlp_body,
        grid=(n // EDGE_BLOCK,),
        in_specs=[
            pl.BlockSpec((EDGE_BLOCK, D), er_map),
            pl.BlockSpec((EDGE_BLOCK, D), row),
            pl.BlockSpec((2 * D, 2 * D), full),
            pl.BlockSpec((1, 2 * D), full),
            pl.BlockSpec((2 * D, D), full),
            pl.BlockSpec((1, D), full),
        ],
        out_specs=pl.BlockSpec((EDGE_BLOCK, D), row),
        out_shape=jax.ShapeDtypeStruct((n, D), jnp.float32),
    )(edge_rep, n2e_p, W1, bias1.reshape(1, -1), W2, bias2.reshape(1, -1))


N_PART = 2 * N_PHASES


def _node_mlp_body(*refs):
    nr_ref = refs[0]
    extras = refs[1:1 + N_PART]
    w1_ref, b1_ref, w2_ref, b2_ref, out_ref = refs[1 + N_PART:]
    e2n = extras[0][...]
    for e in extras[1:]:
        e2n = e2n + e[...]
    x = jnp.concatenate([nr_ref[...].astype(jnp.bfloat16),
                         e2n.astype(jnp.bfloat16)], axis=-1)
    h = jnp.dot(x, w1_ref[...].astype(jnp.bfloat16),
                preferred_element_type=jnp.float32)
    h = jnp.maximum(h + b1_ref[...], 0.0).astype(jnp.bfloat16)
    o = jnp.dot(h, w2_ref[...].astype(jnp.bfloat16),
                preferred_element_type=jnp.float32)
    out_ref[...] = jnp.maximum(o + b2_ref[...], 0.0)


def _node_mlp(node_rep, partials, W1, bias1, W2, bias2):
    row = lambda i: (i, 0)
    full = lambda i: (0, 0)
    return pl.pallas_call(
        _node_mlp_body,
        grid=(N_NODES // NODE_BLOCK,),
        in_specs=[pl.BlockSpec((NODE_BLOCK, D), row)] * (1 + N_PART) + [
            pl.BlockSpec((2 * D, 2 * D), full),
            pl.BlockSpec((1, 2 * D), full),
            pl.BlockSpec((2 * D, D), full),
            pl.BlockSpec((1, D), full),
        ],
        out_specs=pl.BlockSpec((NODE_BLOCK, D), row),
        out_shape=jax.ShapeDtypeStruct((N_NODES, D), jnp.float32),
    )(node_rep, *partials, W1, bias1.reshape(1, -1), W2, bias2.reshape(1, -1))


def _copy_first_body(src_ref, out_ref):
    out_ref[...] = src_ref[...]


def _copy_next_body(prev_ref, src_ref, out_ref):
    out_ref[...] = src_ref[...]


def _assemble_edge_out(prev, eo_p, blk_off):
    nblk = eo_p.shape[0] // EDGE_BLOCK
    row = lambda i: (i, 0)
    out_map = lambda i: (i + blk_off, 0)
    out_shape = jax.ShapeDtypeStruct((N_EDGES, D), jnp.float32)
    if prev is None:
        return pl.pallas_call(
            _copy_first_body,
            grid=(nblk,),
            in_specs=[pl.BlockSpec((EDGE_BLOCK, D), row)],
            out_specs=pl.BlockSpec((EDGE_BLOCK, D), out_map),
            out_shape=out_shape,
        )(eo_p)
    return pl.pallas_call(
        _copy_next_body,
        grid=(nblk,),
        in_specs=[
            pl.BlockSpec((8, D), lambda i: (0, 0)),
            pl.BlockSpec((EDGE_BLOCK, D), row),
        ],
        out_specs=pl.BlockSpec((EDGE_BLOCK, D), out_map),
        out_shape=out_shape,
        input_output_aliases={0: 0},
    )(prev, eo_p)


def kernel(node_rep, edge_rep, edge_index, We1, be1, We2, be2, Wn1, bn1, Wn2, bn2):
    src = edge_index[0]
    dst = edge_index[1]
    zeros = jnp.zeros((N_NODES, D), jnp.float32)

    bounds = [0]
    for e in E_PHASES:
        bounds.append(bounds[-1] + e)

    ident = (jnp.arange(NUM_SUB * NB_G, dtype=jnp.int32)[:, None] * CHUNK
             + jnp.arange(CHUNK, dtype=jnp.int32)[None, :]).reshape(-1)

    gathered = [_sc_gather(node_rep, src, dst, ident, NC_PHASES[p], bounds[p])
                for p in range(N_PHASES)]

    eo_slices = []
    for p in range(N_PHASES):
        eo_slices.append(_edge_mlp_phase(edge_rep, gathered[p], We1, be1,
                                         We2, be2, bounds[p] // EDGE_BLOCK))

    partials = []
    for p in range(N_PHASES):
        part = _sc_scatter(eo_slices[p], src, dst, zeros, NC_PHASES[p],
                           bounds[p])
        partials.extend([part[0], part[1]])

    edge_out = jnp.concatenate(eo_slices, axis=0)
    node_out = _node_mlp(node_rep, partials, Wn1, bn1, Wn2, bn2)
    return (node_out, edge_out)
